# baseline (speedup 1.0000x reference)
import functools

import jax
import jax.numpy as jnp
from jax import lax
from jax.experimental import pallas as pl
from jax.experimental.pallas import tpu as pltpu
from jax.experimental.pallas import tpu_sc as plsc

NC = 2
NS = 16
NW = NC * NS
CH = 128


def _mesh():
    return plsc.VectorSubcoreMesh(core_axis_name="c", subcore_axis_name="s",
                                  num_cores=NC, num_subcores=NS)


_SC_PARAMS = pltpu.CompilerParams(needs_layout_passes=False,
                                  use_tc_tiling_on_sc=False)


def _make_sc_deg(n_pad, ep):
    cpw = ep // (NW * CH)
    rpt = n_pad // NS

    @functools.partial(
        pl.kernel,
        out_type=jax.ShapeDtypeStruct((NC * n_pad,), jnp.float32),
        mesh=_mesh(),
        scratch_types=[
            pltpu.VMEM((cpw, CH), jnp.int32),
            pltpu.VMEM((cpw, CH), jnp.float32),
            pltpu.VMEM((rpt,), jnp.float32),
            pltpu.VMEM_SHARED((n_pad,), jnp.float32),
            pltpu.SemaphoreType.DMA,
        ],
        compiler_params=_SC_PARAMS,
    )
    def deg_kernel(dst_hbm, w_hbm, zeros_hbm, out_hbm, dst_v, w_v, stage_v,
                   acc_sh, ssem):
        c = lax.axis_index("c")
        s = lax.axis_index("s")
        wid = s * NC + c
        sl_rows = pl.ds(s * rpt, rpt)
        pltpu.sync_copy(zeros_hbm.at[sl_rows], stage_v)
        pltpu.sync_copy(stage_v, acc_sh.at[sl_rows])
        base = wid * cpw
        pltpu.sync_copy(dst_hbm.at[pl.ds(base, cpw)], dst_v)
        pltpu.sync_copy(w_hbm.at[pl.ds(base, cpw)], w_v)
        plsc.subcore_barrier()
        for g0 in range(0, cpw, 9):
            descs = [
                pltpu.async_copy(w_v.at[u], acc_sh.at[dst_v.at[u]], ssem,
                                 add=True)
                for u in range(g0, min(g0 + 9, cpw))
            ]
            for dsc in descs:
                dsc.wait()
        plsc.subcore_barrier()
        pltpu.sync_copy(acc_sh.at[sl_rows], stage_v)
        pltpu.sync_copy(stage_v, out_hbm.at[pl.ds(c * n_pad + s * rpt, rpt)])

    return deg_kernel


def _make_sc_agg(n, n_pad, ep, d):
    cpw = ep // (NW * CH)
    rpt = n_pad // NS
    bpb = next(b for b in (9, 8, 7, 6, 5, 4, 3, 2, 1) if cpw % b == 0)
    nblk = cpw // bpb

    @functools.partial(
        pl.kernel,
        out_type=jax.ShapeDtypeStruct((NC, n_pad, d), jnp.float32),
        mesh=_mesh(),
        scratch_types=[
            pltpu.VMEM((bpb, CH), jnp.int32),
            pltpu.VMEM((bpb, CH), jnp.int32),
            pltpu.VMEM((bpb, CH), jnp.float32),
            pltpu.VMEM((CH, d), jnp.float32),
            pltpu.VMEM((CH, d), jnp.float32),
            pltpu.VMEM_SHARED((n_pad, d), jnp.float32),
            pltpu.SemaphoreType.DMA,
            pltpu.SemaphoreType.DMA,
            pltpu.SemaphoreType.DMA,
            pltpu.SemaphoreType.DMA,
        ],
        compiler_params=_SC_PARAMS,
    )
    def agg_kernel(src_hbm, dst_hbm, w_hbm, h_hbm, zeros_hbm,
                   out_hbm, src_v, dst_v, w_v, rows0, rows1,
                   acc_sh, sg0, sg1, ss0, ss1):
        c = lax.axis_index("c")
        s = lax.axis_index("s")
        wid = s * NC + c
        for off in range(0, rpt, CH):
            p = min(CH, rpt - off)
            pltpu.sync_copy(zeros_hbm.at[pl.ds(s * rpt + off, p)],
                            rows0.at[pl.ds(0, p)])
            pltpu.sync_copy(rows0.at[pl.ds(0, p)],
                            acc_sh.at[pl.ds(s * rpt + off, p)])
        plsc.subcore_barrier()

        rows = (rows0, rows1)
        sg = (sg0, sg1)
        ss = (ss0, ss1)
        base = wid * cpw

        def compute(u, p):
            def group(g, gc):
                w16 = w_v[u, pl.ds(g * 16, 16)]

                def edge(e, ec):
                    lane = jnp.zeros((16,), jnp.int32) + e
                    spl = jnp.take_along_axis(w16, lane, axis=0)
                    row = g * 16 + e
                    for k in range(d // 16):
                        cs = pl.ds(k * 16, 16)
                        rows[p][row, cs] = rows[p][row, cs] * spl
                    return ec

                lax.fori_loop(0, 16, edge, 0)
                return gc

            lax.fori_loop(0, CH // 16, group, 0)

        def block(b, carry):
            br = base + b * bpb
            pltpu.sync_copy(src_hbm.at[pl.ds(br, bpb)], src_v)
            pltpu.sync_copy(dst_hbm.at[pl.ds(br, bpb)], dst_v)
            pltpu.sync_copy(w_hbm.at[pl.ds(br, bpb)], w_v)
            gath = [None] * bpb
            scat = [None] * bpb
            gath[0] = pltpu.async_copy(h_hbm.at[src_v.at[0]], rows[0], sg[0])
            for u in range(bpb):
                p = u % 2
                if u + 1 < bpb:
                    if u >= 1:
                        scat[u - 1].wait()
                    gath[u + 1] = pltpu.async_copy(
                        h_hbm.at[src_v.at[u + 1]], rows[1 - p], sg[1 - p])
                gath[u].wait()
                compute(u, p)
                scat[u] = pltpu.async_copy(
                    rows[p], acc_sh.at[dst_v.at[u]], ss[p], add=True)
            if bpb >= 2:
                scat[bpb - 2].wait()
            scat[bpb - 1].wait()
            return carry

        lax.fori_loop(0, nblk, block, 0)
        plsc.subcore_barrier()
        for off in range(0, rpt, CH):
            p = min(CH, rpt - off)
            pltpu.sync_copy(acc_sh.at[pl.ds(s * rpt + off, p)],
                            rows0.at[pl.ds(0, p)])
            pltpu.sync_copy(rows0.at[pl.ds(0, p)],
                            out_hbm.at[c, pl.ds(s * rpt + off, p)])

    return agg_kernel


def _tc_mm_dis_body(x_ref, w_ref, degp_ref, o_ref, dis_ref):
    i = pl.program_id(0)

    @pl.when(i == 0)
    def _():
        dg = degp_ref[0] + degp_ref[1]
        dis_ref[...] = jnp.where(
            dg > 0, lax.rsqrt(jnp.maximum(dg, 1e-12)), 0.0)

    blk = x_ref.shape[0]
    dis_blk = dis_ref[pl.ds(i * blk, blk), :]
    o_ref[...] = dis_blk * lax.dot_general(
        x_ref[...], w_ref[...], (((1,), (1,)), ((), ())),
        preferred_element_type=jnp.float32)


def _tc_mm_dis(x, w, deg_parts, blk):
    n, f = x.shape
    d = w.shape[0]
    n_pad = deg_parts.shape[1]
    return pl.pallas_call(
        _tc_mm_dis_body,
        grid=(n // blk,),
        in_specs=[pl.BlockSpec((blk, f), lambda i: (i, 0)),
                  pl.BlockSpec((d, f), lambda i: (0, 0)),
                  pl.BlockSpec((NC, n_pad, 1), lambda i: (0, 0, 0))],
        out_specs=[pl.BlockSpec((blk, d), lambda i: (i, 0)),
                   pl.BlockSpec((n_pad, 1), lambda i: (0, 0))],
        out_shape=[jax.ShapeDtypeStruct((n, d), jnp.float32),
                   jax.ShapeDtypeStruct((n_pad, 1), jnp.float32)],
    )(x, w, deg_parts)


def _tc_epi1_body(agg_ref, dis_ref, b_ref, w2_ref, o_ref):
    a = dis_ref[...] * (agg_ref[0] + agg_ref[1]) + b_ref[...]
    r = jnp.maximum(a, 0.0)
    o_ref[...] = dis_ref[...] * lax.dot_general(
        r, w2_ref[...], (((1,), (1,)), ((), ())),
        preferred_element_type=jnp.float32)


def _tc_epi1(agg_parts, dis, b1, w2, blk):
    n_pad, d1 = agg_parts.shape[1], agg_parts.shape[2]
    d2 = w2.shape[0]
    return pl.pallas_call(
        _tc_epi1_body,
        grid=(n_pad // blk,),
        in_specs=[pl.BlockSpec((NC, blk, d1), lambda i: (0, i, 0)),
                  pl.BlockSpec((blk, 1), lambda i: (i, 0)),
                  pl.BlockSpec((d1,), lambda i: (0,)),
                  pl.BlockSpec((d2, d1), lambda i: (0, 0))],
        out_specs=pl.BlockSpec((blk, d2), lambda i: (i, 0)),
        out_shape=jax.ShapeDtypeStruct((n_pad, d2), jnp.float32),
    )(agg_parts, dis, b1, w2)


def _tc_epi2_body(agg_ref, dis_ref, b_ref, o_ref):
    o_ref[...] = dis_ref[...] * (agg_ref[0] + agg_ref[1]) + b_ref[...]


def _tc_epi2(agg_parts, dis, b2, blk):
    n_pad, d2 = agg_parts.shape[1], agg_parts.shape[2]
    return pl.pallas_call(
        _tc_epi2_body,
        grid=(n_pad // blk,),
        in_specs=[pl.BlockSpec((NC, blk, d2), lambda i: (0, i, 0)),
                  pl.BlockSpec((blk, 1), lambda i: (i, 0)),
                  pl.BlockSpec((d2,), lambda i: (0,))],
        out_specs=pl.BlockSpec((blk, d2), lambda i: (i, 0)),
        out_shape=jax.ShapeDtypeStruct((n_pad, d2), jnp.float32),
    )(agg_parts, dis, b2)


def kernel(x, edge_index, edge_weight, W1, b1, W2, b2):
    n, _ = x.shape
    e = edge_index.shape[1]
    d1 = W1.shape[0]
    d2 = W2.shape[0]
    grain = NW * CH
    ep = ((e + n + grain - 1) // grain) * grain
    pad = ep - (e + n)
    rpt = ((n + NS - 1) // NS + 7) // 8 * 8
    n_pad = NS * rpt

    loop = jnp.arange(n, dtype=jnp.int32)
    zpad_i = jnp.zeros((pad,), jnp.int32)
    src_f = jnp.concatenate([edge_index[0], loop, zpad_i])
    dst_f = jnp.concatenate([edge_index[1], loop, zpad_i])
    w_f = jnp.concatenate([edge_weight, jnp.ones((n,), jnp.float32),
                           jnp.zeros((pad,), jnp.float32)])
    src2 = src_f.reshape(ep // CH, CH)
    dst2 = dst_f.reshape(ep // CH, CH)
    w2 = w_f.reshape(ep // CH, CH)

    zeros1 = jnp.zeros((n_pad,), jnp.float32)
    zeros_d1 = jnp.zeros((n_pad, d1), jnp.float32)
    zeros_d2 = jnp.zeros((n_pad, d2), jnp.float32)

    deg_flat = _make_sc_deg(n_pad, ep)(dst2, w2, zeros1)
    h1p, dis = _tc_mm_dis(x, W1, deg_flat.reshape(NC, n_pad, 1), n // 10)

    agg1 = _make_sc_agg(n, n_pad, ep, d1)(src2, dst2, w2, h1p, zeros_d1)
    h2p = _tc_epi1(agg1, dis, b1, W2, rpt)
    agg2 = _make_sc_agg(n, n_pad, ep, d2)(src2, dst2, w2, h2p, zeros_d2)
    return _tc_epi2(agg2, dis, b2, rpt)[:n]


# speedup vs baseline: 1.0727x; 1.0727x over previous
import functools

import jax
import jax.numpy as jnp
from jax import lax
from jax.experimental import pallas as pl
from jax.experimental.pallas import tpu as pltpu
from jax.experimental.pallas import tpu_sc as plsc

NC = 2
NS = 16
NW = NC * NS
CH = 128


def _mesh():
    return plsc.VectorSubcoreMesh(core_axis_name="c", subcore_axis_name="s",
                                  num_cores=NC, num_subcores=NS)


_SC_PARAMS = pltpu.CompilerParams(needs_layout_passes=False,
                                  use_tc_tiling_on_sc=False)


def _make_sc_deg(n_pad, ep):
    cpw = ep // (NW * CH)
    rpt = n_pad // NS

    @functools.partial(
        pl.kernel,
        out_type=jax.ShapeDtypeStruct((NC * n_pad,), jnp.float32),
        mesh=_mesh(),
        scratch_types=[
            pltpu.VMEM((cpw, CH), jnp.int32),
            pltpu.VMEM((cpw, CH), jnp.float32),
            pltpu.VMEM((rpt,), jnp.float32),
            pltpu.VMEM_SHARED((n_pad,), jnp.float32),
            pltpu.SemaphoreType.DMA,
        ],
        compiler_params=_SC_PARAMS,
    )
    def deg_kernel(dst_hbm, w_hbm, zeros_hbm, out_hbm, dst_v, w_v, stage_v,
                   acc_sh, ssem):
        c = lax.axis_index("c")
        s = lax.axis_index("s")
        wid = s * NC + c
        sl_rows = pl.ds(s * rpt, rpt)
        pltpu.sync_copy(zeros_hbm.at[sl_rows], stage_v)
        pltpu.sync_copy(stage_v, acc_sh.at[sl_rows])
        base = wid * cpw
        pltpu.sync_copy(dst_hbm.at[pl.ds(base, cpw)], dst_v)
        pltpu.sync_copy(w_hbm.at[pl.ds(base, cpw)], w_v)
        plsc.subcore_barrier()
        for g0 in range(0, cpw, 9):
            descs = [
                pltpu.async_copy(w_v.at[u], acc_sh.at[dst_v.at[u]], ssem,
                                 add=True)
                for u in range(g0, min(g0 + 9, cpw))
            ]
            for dsc in descs:
                dsc.wait()
        plsc.subcore_barrier()
        pltpu.sync_copy(acc_sh.at[sl_rows], stage_v)
        pltpu.sync_copy(stage_v, out_hbm.at[pl.ds(c * n_pad + s * rpt, rpt)])

    return deg_kernel


def _make_sc_agg(n, n_pad, ep, d):
    cpw = ep // (NW * CH)
    rpt = n_pad // NS
    def _fits(b):
        words = NS * (3 * b * CH + 2 * CH * d) + n_pad * d
        return cpw % b == 0 and words <= 2_000_000
    bpb = next(b for b in (81, 27, 9, 8, 7, 6, 5, 4, 3, 2, 1) if _fits(b))
    nblk = cpw // bpb

    @functools.partial(
        pl.kernel,
        out_type=jax.ShapeDtypeStruct((NC, n_pad, d), jnp.float32),
        mesh=_mesh(),
        scratch_types=[
            pltpu.VMEM((bpb, CH), jnp.int32),
            pltpu.VMEM((bpb, CH), jnp.int32),
            pltpu.VMEM((bpb, CH), jnp.float32),
            pltpu.VMEM((CH, d), jnp.float32),
            pltpu.VMEM((CH, d), jnp.float32),
            pltpu.VMEM_SHARED((n_pad, d), jnp.float32),
            pltpu.SemaphoreType.DMA,
            pltpu.SemaphoreType.DMA,
            pltpu.SemaphoreType.DMA,
            pltpu.SemaphoreType.DMA,
        ],
        compiler_params=_SC_PARAMS,
    )
    def agg_kernel(src_hbm, dst_hbm, w_hbm, h_hbm, zeros_hbm,
                   out_hbm, src_v, dst_v, w_v, rows0, rows1,
                   acc_sh, sg0, sg1, ss0, ss1):
        c = lax.axis_index("c")
        s = lax.axis_index("s")
        wid = s * NC + c
        for off in range(0, rpt, CH):
            p = min(CH, rpt - off)
            pltpu.sync_copy(zeros_hbm.at[pl.ds(s * rpt + off, p)],
                            rows0.at[pl.ds(0, p)])
            pltpu.sync_copy(rows0.at[pl.ds(0, p)],
                            acc_sh.at[pl.ds(s * rpt + off, p)])
        plsc.subcore_barrier()

        rows = (rows0, rows1)
        sg = (sg0, sg1)
        ss = (ss0, ss1)
        base = wid * cpw

        def compute(u, p):
            def group(g, gc):
                w16 = w_v[u, pl.ds(g * 16, 16)]

                def edge(e, ec):
                    lane = jnp.zeros((16,), jnp.int32) + e
                    spl = jnp.take_along_axis(w16, lane, axis=0)
                    row = g * 16 + e
                    for k in range(d // 16):
                        cs = pl.ds(k * 16, 16)
                        rows[p][row, cs] = rows[p][row, cs] * spl
                    return ec

                lax.fori_loop(0, 16, edge, 0)
                return gc

            lax.fori_loop(0, CH // 16, group, 0)

        def block(b, carry):
            br = base + b * bpb
            pltpu.sync_copy(src_hbm.at[pl.ds(br, bpb)], src_v)
            pltpu.sync_copy(dst_hbm.at[pl.ds(br, bpb)], dst_v)
            pltpu.sync_copy(w_hbm.at[pl.ds(br, bpb)], w_v)
            gath = [None] * bpb
            scat = [None] * bpb
            gath[0] = pltpu.async_copy(h_hbm.at[src_v.at[0]], rows[0], sg[0])
            for u in range(bpb):
                p = u % 2
                if u + 1 < bpb:
                    if u >= 1:
                        scat[u - 1].wait()
                    gath[u + 1] = pltpu.async_copy(
                        h_hbm.at[src_v.at[u + 1]], rows[1 - p], sg[1 - p])
                gath[u].wait()
                compute(u, p)
                scat[u] = pltpu.async_copy(
                    rows[p], acc_sh.at[dst_v.at[u]], ss[p], add=True)
            if bpb >= 2:
                scat[bpb - 2].wait()
            scat[bpb - 1].wait()
            return carry

        lax.fori_loop(0, nblk, block, 0)
        plsc.subcore_barrier()
        for off in range(0, rpt, CH):
            p = min(CH, rpt - off)
            pltpu.sync_copy(acc_sh.at[pl.ds(s * rpt + off, p)],
                            rows0.at[pl.ds(0, p)])
            pltpu.sync_copy(rows0.at[pl.ds(0, p)],
                            out_hbm.at[c, pl.ds(s * rpt + off, p)])

    return agg_kernel


def _tc_mm_dis_body(x_ref, w_ref, degp_ref, o_ref, dis_ref):
    i = pl.program_id(0)

    @pl.when(i == 0)
    def _():
        dg = degp_ref[0] + degp_ref[1]
        dis_ref[...] = jnp.where(
            dg > 0, lax.rsqrt(jnp.maximum(dg, 1e-12)), 0.0)

    blk = x_ref.shape[0]
    dis_blk = dis_ref[pl.ds(i * blk, blk), :]
    o_ref[...] = dis_blk * lax.dot_general(
        x_ref[...], w_ref[...], (((1,), (1,)), ((), ())),
        preferred_element_type=jnp.float32)


def _tc_mm_dis(x, w, deg_parts, blk):
    n, f = x.shape
    d = w.shape[0]
    n_pad = deg_parts.shape[1]
    return pl.pallas_call(
        _tc_mm_dis_body,
        grid=(n // blk,),
        in_specs=[pl.BlockSpec((blk, f), lambda i: (i, 0)),
                  pl.BlockSpec((d, f), lambda i: (0, 0)),
                  pl.BlockSpec((NC, n_pad, 1), lambda i: (0, 0, 0))],
        out_specs=[pl.BlockSpec((blk, d), lambda i: (i, 0)),
                   pl.BlockSpec((n_pad, 1), lambda i: (0, 0))],
        out_shape=[jax.ShapeDtypeStruct((n, d), jnp.float32),
                   jax.ShapeDtypeStruct((n_pad, 1), jnp.float32)],
    )(x, w, deg_parts)


def _tc_epi1_body(agg_ref, dis_ref, b_ref, w2_ref, o_ref):
    a = dis_ref[...] * (agg_ref[0] + agg_ref[1]) + b_ref[...]
    r = jnp.maximum(a, 0.0)
    o_ref[...] = dis_ref[...] * lax.dot_general(
        r, w2_ref[...], (((1,), (1,)), ((), ())),
        preferred_element_type=jnp.float32)


def _tc_epi1(agg_parts, dis, b1, w2, blk):
    n_pad, d1 = agg_parts.shape[1], agg_parts.shape[2]
    d2 = w2.shape[0]
    return pl.pallas_call(
        _tc_epi1_body,
        grid=(n_pad // blk,),
        in_specs=[pl.BlockSpec((NC, blk, d1), lambda i: (0, i, 0)),
                  pl.BlockSpec((blk, 1), lambda i: (i, 0)),
                  pl.BlockSpec((d1,), lambda i: (0,)),
                  pl.BlockSpec((d2, d1), lambda i: (0, 0))],
        out_specs=pl.BlockSpec((blk, d2), lambda i: (i, 0)),
        out_shape=jax.ShapeDtypeStruct((n_pad, d2), jnp.float32),
    )(agg_parts, dis, b1, w2)


def _tc_epi2_body(agg_ref, dis_ref, b_ref, o_ref):
    o_ref[...] = dis_ref[...] * (agg_ref[0] + agg_ref[1]) + b_ref[...]


def _tc_epi2(agg_parts, dis, b2, blk):
    n_pad, d2 = agg_parts.shape[1], agg_parts.shape[2]
    return pl.pallas_call(
        _tc_epi2_body,
        grid=(n_pad // blk,),
        in_specs=[pl.BlockSpec((NC, blk, d2), lambda i: (0, i, 0)),
                  pl.BlockSpec((blk, 1), lambda i: (i, 0)),
                  pl.BlockSpec((d2,), lambda i: (0,))],
        out_specs=pl.BlockSpec((blk, d2), lambda i: (i, 0)),
        out_shape=jax.ShapeDtypeStruct((n_pad, d2), jnp.float32),
    )(agg_parts, dis, b2)


def kernel(x, edge_index, edge_weight, W1, b1, W2, b2):
    n, _ = x.shape
    e = edge_index.shape[1]
    d1 = W1.shape[0]
    d2 = W2.shape[0]
    grain = NW * CH
    ep = ((e + n + grain - 1) // grain) * grain
    pad = ep - (e + n)
    rpt = ((n + NS - 1) // NS + 7) // 8 * 8
    n_pad = NS * rpt

    loop = jnp.arange(n, dtype=jnp.int32)
    zpad_i = jnp.zeros((pad,), jnp.int32)
    src_f = jnp.concatenate([edge_index[0], loop, zpad_i])
    dst_f = jnp.concatenate([edge_index[1], loop, zpad_i])
    w_f = jnp.concatenate([edge_weight, jnp.ones((n,), jnp.float32),
                           jnp.zeros((pad,), jnp.float32)])
    src2 = src_f.reshape(ep // CH, CH)
    dst2 = dst_f.reshape(ep // CH, CH)
    w2 = w_f.reshape(ep // CH, CH)

    zeros1 = jnp.zeros((n_pad,), jnp.float32)
    zeros_d1 = jnp.zeros((n_pad, d1), jnp.float32)
    zeros_d2 = jnp.zeros((n_pad, d2), jnp.float32)

    deg_flat = _make_sc_deg(n_pad, ep)(dst2, w2, zeros1)
    h1p, dis = _tc_mm_dis(x, W1, deg_flat.reshape(NC, n_pad, 1), n // 10)

    agg1 = _make_sc_agg(n, n_pad, ep, d1)(src2, dst2, w2, h1p, zeros_d1)
    h2p = _tc_epi1(agg1, dis, b1, W2, rpt)
    agg2 = _make_sc_agg(n, n_pad, ep, d2)(src2, dst2, w2, h2p, zeros_d2)
    return _tc_epi2(agg2, dis, b2, rpt)[:n]
